# baseline (speedup 1.0000x reference)
import functools

import jax
import jax.numpy as jnp
from jax import lax
from jax.experimental import pallas as pl
from jax.experimental.pallas import tpu as pltpu
from jax.experimental.pallas import tpu_sc as plsc

N_NODES = 10000
N_EDGES = 320000
EMBED = 128
N_QUERY = 4096

E_BLOCK = 2048

NC = 2
NS = 16
NW = NC * NS
EDGES_PER_TILE = N_EDGES // NW
CHUNK = 80
NCHUNK = EDGES_PER_TILE // CHUNK
N_NODES_PAD = 10240
ZROWS = N_NODES_PAD // NS
QC = 64
NQC = N_QUERY // QC
Q_PER_TILE = N_QUERY // NS


def _mlp_body(u_ref, rep_ref, w1_ref, b1_ref, w2_ref, b2_ref,
              w3_ref, b3_ref, p_ref):
    u = u_ref[...]
    x = jnp.dot(u, w1_ref[0:EMBED], preferred_element_type=jnp.float32)
    x = x + jnp.dot(rep_ref[...], w1_ref[EMBED:2 * EMBED],
                    preferred_element_type=jnp.float32)
    x = jnp.maximum(x + b1_ref[...], 0.0)
    x = jnp.maximum(jnp.dot(x, w2_ref[...], preferred_element_type=jnp.float32)
                    + b2_ref[...], 0.0)
    s = jnp.dot(x, w3_ref[...], preferred_element_type=jnp.float32) + b3_ref[...]
    p = jnp.exp(s)
    p_ref[...] = p


def _mlp_stage(u, rep, w1, b1, w2, b2, w3, b3r):
    grid = (N_EDGES + E_BLOCK - 1) // E_BLOCK
    return pl.pallas_call(
        _mlp_body,
        grid=(grid,),
        in_specs=[
            pl.BlockSpec((E_BLOCK, EMBED), lambda i: (i, 0)),
            pl.BlockSpec((E_BLOCK, EMBED), lambda i: (i, 0)),
            pl.BlockSpec((2 * EMBED, EMBED), lambda i: (0, 0)),
            pl.BlockSpec((1, EMBED), lambda i: (0, 0)),
            pl.BlockSpec((EMBED, EMBED), lambda i: (0, 0)),
            pl.BlockSpec((1, EMBED), lambda i: (0, 0)),
            pl.BlockSpec((EMBED, 1), lambda i: (0, 0)),
            pl.BlockSpec((1, 1), lambda i: (0, 0)),
        ],
        out_specs=pl.BlockSpec((E_BLOCK, 1), lambda i: (i, 0)),
        out_shape=jax.ShapeDtypeStruct((N_EDGES, 1), jnp.float32),
    )(u, rep, w1, b1, w2, b2, w3, b3r)


def _sc_body(h_hbm, p_hbm, dst_hbm, nodes_hbm, num_hbm, den_hbm,
             h0_v, h1_v, d0_v, d1_v, p0_v, p1_v, denom_v, qidx_v, qden_v, z_sh,
             hsem0, hsem1, dsem0, dsem1, psem0, psem1):
    cid = lax.axis_index("c")
    sid = lax.axis_index("s")
    wid = sid * NC + cid

    zeros16 = jnp.zeros((16,), jnp.float32)

    def zero_row(r, _):
        for k in range(EMBED // 16):
            h0_v[r, pl.ds(k * 16, 16)] = zeros16
        return 0
    lax.fori_loop(0, CHUNK, zero_row, 0)

    def zero_den(i, _):
        denom_v[pl.ds(i * 16, 16)] = zeros16
        return 0
    lax.fori_loop(0, N_NODES_PAD // 16, zero_den, 0)

    def zcopy(j, _):
        pltpu.sync_copy(h0_v, z_sh.at[pl.ds(sid * ZROWS + j * CHUNK, CHUNK)])
        return 0
    lax.fori_loop(0, ZROWS // CHUNK, zcopy, 0)

    plsc.subcore_barrier()

    def _start_load(j, hbuf, dbuf, pbuf, hsem, dsem, psem):
        base = wid * EDGES_PER_TILE + j * CHUNK
        pltpu.async_copy(h_hbm.at[pl.ds(base, CHUNK)], hbuf, hsem)
        pltpu.async_copy(dst_hbm.at[pl.ds(base, CHUNK)], dbuf, dsem)
        pltpu.async_copy(p_hbm.at[pl.ds(base, CHUNK)], pbuf, psem)

    def _wait_load(j, hbuf, dbuf, pbuf, hsem, dsem, psem):
        base = wid * EDGES_PER_TILE + j * CHUNK
        pltpu.make_async_copy(h_hbm.at[pl.ds(base, CHUNK)], hbuf, hsem).wait()
        pltpu.make_async_copy(dst_hbm.at[pl.ds(base, CHUNK)], dbuf, dsem).wait()
        pltpu.make_async_copy(p_hbm.at[pl.ds(base, CHUNK)], pbuf, psem).wait()

    def _process(j, hbuf, dbuf, pbuf):
        zi16 = jnp.zeros((16,), jnp.int32)

        def scale_row(r, _):
            pb = plsc.load_gather(pbuf, [jnp.full((16,), r, jnp.int32), zi16])
            for k in range(EMBED // 16):
                hbuf[r, pl.ds(k * 16, 16)] = hbuf[r, pl.ds(k * 16, 16)] * pb
            return 0
        lax.fori_loop(0, CHUNK, scale_row, 0)
        pltpu.sync_copy(hbuf, z_sh.at[dbuf], add=True)
        for k in range(CHUNK // 16):
            dvec = dbuf[pl.ds(k * 16, 16)]
            ivec = jnp.arange(16, dtype=jnp.int32) + (16 * k)
            pvec = plsc.load_gather(pbuf, [ivec, zi16])
            plsc.addupdate_scatter(denom_v, [dvec], pvec)

    _start_load(0, h0_v, d0_v, p0_v, hsem0, dsem0, psem0)

    def chunk_pair(j2, _):
        j0 = 2 * j2
        _wait_load(j0, h0_v, d0_v, p0_v, hsem0, dsem0, psem0)
        _start_load(j0 + 1, h1_v, d1_v, p1_v, hsem1, dsem1, psem1)
        _process(j0, h0_v, d0_v, p0_v)
        _wait_load(j0 + 1, h1_v, d1_v, p1_v, hsem1, dsem1, psem1)
        _start_load(j0 + 2, h0_v, d0_v, p0_v, hsem0, dsem0, psem0)
        _process(j0 + 1, h1_v, d1_v, p1_v)
        return 0
    lax.fori_loop(0, (NCHUNK - 1) // 2, chunk_pair, 0)
    _wait_load(NCHUNK - 1, h0_v, d0_v, p0_v, hsem0, dsem0, psem0)
    _process(NCHUNK - 1, h0_v, d0_v, p0_v)

    def qden(t, _):
        pltpu.sync_copy(nodes_hbm.at[pl.ds(t * QC, QC)], qidx_v)
        for k in range(QC // 16):
            ivec = qidx_v[pl.ds(k * 16, 16)]
            qden_v[pl.ds(t * QC + k * 16, 16)] = plsc.load_gather(
                denom_v, [ivec])
        return 0
    lax.fori_loop(0, NQC, qden, 0)
    pltpu.sync_copy(qden_v, den_hbm.at[cid, sid])

    plsc.subcore_barrier()

    def qchunk(t, _):
        row = sid * (Q_PER_TILE // QC) + t
        pltpu.sync_copy(nodes_hbm.at[pl.ds(row * QC, QC)], qidx_v)
        pltpu.sync_copy(z_sh.at[qidx_v], h0_v.at[pl.ds(0, QC)])
        pltpu.sync_copy(h0_v.at[pl.ds(0, QC)],
                        num_hbm.at[cid, pl.ds(row * QC, QC)])
        return 0
    lax.fori_loop(0, Q_PER_TILE // QC, qchunk, 0)


def _sc_stage(h, p, dst, nodes):
    mesh = plsc.VectorSubcoreMesh(core_axis_name="c", subcore_axis_name="s")
    f = functools.partial(
        pl.kernel, mesh=mesh,
        compiler_params=pltpu.CompilerParams(needs_layout_passes=False,
                                             use_tc_tiling_on_sc=False),
        out_type=[
            jax.ShapeDtypeStruct((NC, N_QUERY, EMBED), jnp.float32),
            jax.ShapeDtypeStruct((NC, NS, N_QUERY), jnp.float32),
        ],
        scratch_types=[
            pltpu.VMEM((CHUNK, EMBED), jnp.float32),
            pltpu.VMEM((CHUNK, EMBED), jnp.float32),
            pltpu.VMEM((CHUNK,), jnp.int32),
            pltpu.VMEM((CHUNK,), jnp.int32),
            pltpu.VMEM((CHUNK, 1), jnp.float32),
            pltpu.VMEM((CHUNK, 1), jnp.float32),
            pltpu.VMEM((N_NODES_PAD,), jnp.float32),
            pltpu.VMEM((QC,), jnp.int32),
            pltpu.VMEM((N_QUERY,), jnp.float32),
            pltpu.VMEM_SHARED((N_NODES_PAD, EMBED), jnp.float32),
            pltpu.SemaphoreType.DMA,
            pltpu.SemaphoreType.DMA,
            pltpu.SemaphoreType.DMA,
            pltpu.SemaphoreType.DMA,
            pltpu.SemaphoreType.DMA,
            pltpu.SemaphoreType.DMA,
        ],
    )(_sc_body)
    return f(h, p, dst, nodes)


def _combine_body(num_ref, den_ref, out_ref):
    n = num_ref[0] + num_ref[1]
    d = jnp.sum(den_ref[...], axis=(0, 1))
    out_ref[...] = n / (d[:, None] + 1e-16)


def _combine_stage(num, den):
    QB = 512
    return pl.pallas_call(
        _combine_body,
        grid=(N_QUERY // QB,),
        in_specs=[
            pl.BlockSpec((NC, QB, EMBED), lambda i: (0, i, 0)),
            pl.BlockSpec((NC, NS, QB), lambda i: (0, 0, i)),
        ],
        out_specs=pl.BlockSpec((QB, EMBED), lambda i: (i, 0)),
        out_shape=jax.ShapeDtypeStruct((N_QUERY, EMBED), jnp.float32),
    )(num, den)


def kernel(nodes, edge_index, embed_u, rep, W1, b1, W2, b2, W3, b3):
    dst = edge_index[1]
    b1r = b1.reshape(1, EMBED)
    b2r = b2.reshape(1, EMBED)
    b3r = b3.reshape(1, 1)
    p = _mlp_stage(embed_u, rep, W1, b1r, W2, b2r, W3, b3r)
    num, den = _sc_stage(embed_u, p, dst.astype(jnp.int32),
                         nodes.astype(jnp.int32))
    return _combine_stage(num, den)


# speedup vs baseline: 1.3475x; 1.3475x over previous
import functools

import jax
import jax.numpy as jnp
from jax import lax
from jax.experimental import pallas as pl
from jax.experimental.pallas import tpu as pltpu
from jax.experimental.pallas import tpu_sc as plsc

N_NODES = 10000
N_EDGES = 320000
EMBED = 128
N_QUERY = 4096

E_BLOCK = 2048

NC = 2
NS = 16
NW = NC * NS
EDGES_PER_TILE = N_EDGES // NW
CHUNK = 80
NCHUNK = EDGES_PER_TILE // CHUNK
N_NODES_PAD = 10240
ZROWS = N_NODES_PAD // NS
QC = 64
NQC = N_QUERY // QC
Q_PER_TILE = N_QUERY // NS


def _mlp_body(u_ref, rep_ref, w1_ref, b1_ref, w2_ref, b2_ref,
              w3_ref, b3_ref, h_ref, p_ref):
    u = u_ref[...]
    x = jnp.dot(u, w1_ref[0:EMBED], preferred_element_type=jnp.float32)
    x = x + jnp.dot(rep_ref[...], w1_ref[EMBED:2 * EMBED],
                    preferred_element_type=jnp.float32)
    x = jnp.maximum(x + b1_ref[...], 0.0)
    x = jnp.maximum(jnp.dot(x, w2_ref[...], preferred_element_type=jnp.float32)
                    + b2_ref[...], 0.0)
    s = jnp.dot(x, w3_ref[...], preferred_element_type=jnp.float32) + b3_ref[...]
    p = jnp.exp(s)
    h_ref[...] = u * p
    p_ref[...] = p[:, 0]


def _mlp_stage(u, rep, w1, b1, w2, b2, w3, b3r):
    grid = (N_EDGES + E_BLOCK - 1) // E_BLOCK
    return pl.pallas_call(
        _mlp_body,
        grid=(grid,),
        in_specs=[
            pl.BlockSpec((E_BLOCK, EMBED), lambda i: (i, 0)),
            pl.BlockSpec((E_BLOCK, EMBED), lambda i: (i, 0)),
            pl.BlockSpec((2 * EMBED, EMBED), lambda i: (0, 0)),
            pl.BlockSpec((1, EMBED), lambda i: (0, 0)),
            pl.BlockSpec((EMBED, EMBED), lambda i: (0, 0)),
            pl.BlockSpec((1, EMBED), lambda i: (0, 0)),
            pl.BlockSpec((EMBED, 1), lambda i: (0, 0)),
            pl.BlockSpec((1, 1), lambda i: (0, 0)),
        ],
        out_specs=[
            pl.BlockSpec((E_BLOCK, EMBED), lambda i: (i, 0)),
            pl.BlockSpec((E_BLOCK,), lambda i: (i,)),
        ],
        out_shape=[
            jax.ShapeDtypeStruct((N_EDGES, EMBED), jnp.float32),
            jax.ShapeDtypeStruct((N_EDGES,), jnp.float32),
        ],
    )(u, rep, w1, b1, w2, b2, w3, b3r)


def _sc_body(h_hbm, p_hbm, dst_hbm, nodes_hbm, num_hbm, den_hbm,
             h0_v, h1_v, d0_v, d1_v, p0_v, p1_v, denom_v, qidx_v, qden_v, z_sh,
             hsem0, hsem1, dsem0, dsem1, psem0, psem1, ssem0, ssem1):
    cid = lax.axis_index("c")
    sid = lax.axis_index("s")
    wid = sid * NC + cid

    zeros16 = jnp.zeros((16,), jnp.float32)

    def zero_row(r, _):
        for k in range(EMBED // 16):
            h0_v[r, pl.ds(k * 16, 16)] = zeros16
        return 0
    lax.fori_loop(0, CHUNK, zero_row, 0)

    def zero_den(i, _):
        denom_v[pl.ds(i * 16, 16)] = zeros16
        return 0
    lax.fori_loop(0, N_NODES_PAD // 16, zero_den, 0)

    def zcopy(j, _):
        pltpu.sync_copy(h0_v, z_sh.at[pl.ds(sid * ZROWS + j * CHUNK, CHUNK)])
        return 0
    lax.fori_loop(0, ZROWS // CHUNK, zcopy, 0)

    plsc.subcore_barrier()

    def _start_load(j, hbuf, dbuf, pbuf, hsem, dsem, psem):
        base = wid * EDGES_PER_TILE + j * CHUNK
        pltpu.async_copy(h_hbm.at[pl.ds(base, CHUNK)], hbuf, hsem)
        pltpu.async_copy(dst_hbm.at[pl.ds(base, CHUNK)], dbuf, dsem)
        pltpu.async_copy(p_hbm.at[pl.ds(base, CHUNK)], pbuf, psem)

    def _wait_load(j, hbuf, dbuf, pbuf, hsem, dsem, psem):
        base = wid * EDGES_PER_TILE + j * CHUNK
        pltpu.make_async_copy(h_hbm.at[pl.ds(base, CHUNK)], hbuf, hsem).wait()
        pltpu.make_async_copy(dst_hbm.at[pl.ds(base, CHUNK)], dbuf, dsem).wait()
        pltpu.make_async_copy(p_hbm.at[pl.ds(base, CHUNK)], pbuf, psem).wait()

    def _start_scatter(hbuf, dbuf, ssem):
        pltpu.async_copy(hbuf, z_sh.at[dbuf], ssem, add=True)

    def _wait_scatter(hbuf, dbuf, ssem):
        pltpu.make_async_copy(hbuf, z_sh.at[dbuf], ssem).wait()

    def _denom(dbuf, pbuf):
        for k in range(CHUNK // 16):
            dvec = dbuf[pl.ds(k * 16, 16)]
            pvec = pbuf[pl.ds(k * 16, 16)]
            plsc.addupdate_scatter(denom_v, [dvec], pvec)

    _start_load(0, h0_v, d0_v, p0_v, hsem0, dsem0, psem0)

    def chunk_pair(j2, _):
        j0 = 2 * j2
        _wait_load(j0, h0_v, d0_v, p0_v, hsem0, dsem0, psem0)

        @pl.when(j2 > 0)
        def _():
            _wait_scatter(h1_v, d1_v, ssem1)
        _start_load(j0 + 1, h1_v, d1_v, p1_v, hsem1, dsem1, psem1)
        _start_scatter(h0_v, d0_v, ssem0)
        _denom(d0_v, p0_v)
        _wait_load(j0 + 1, h1_v, d1_v, p1_v, hsem1, dsem1, psem1)
        _wait_scatter(h0_v, d0_v, ssem0)
        _start_load(j0 + 2, h0_v, d0_v, p0_v, hsem0, dsem0, psem0)
        _start_scatter(h1_v, d1_v, ssem1)
        _denom(d1_v, p1_v)
        return 0
    lax.fori_loop(0, (NCHUNK - 1) // 2, chunk_pair, 0)
    _wait_scatter(h1_v, d1_v, ssem1)
    _wait_load(NCHUNK - 1, h0_v, d0_v, p0_v, hsem0, dsem0, psem0)
    _start_scatter(h0_v, d0_v, ssem0)
    _denom(d0_v, p0_v)
    _wait_scatter(h0_v, d0_v, ssem0)

    def qden(t, _):
        pltpu.sync_copy(nodes_hbm.at[pl.ds(t * QC, QC)], qidx_v)
        for k in range(QC // 16):
            ivec = qidx_v[pl.ds(k * 16, 16)]
            qden_v[pl.ds(t * QC + k * 16, 16)] = plsc.load_gather(
                denom_v, [ivec])
        return 0
    lax.fori_loop(0, NQC, qden, 0)
    pltpu.sync_copy(qden_v, den_hbm.at[cid, sid])

    plsc.subcore_barrier()

    def qchunk(t, _):
        row = sid * (Q_PER_TILE // QC) + t
        pltpu.sync_copy(nodes_hbm.at[pl.ds(row * QC, QC)], qidx_v)
        pltpu.sync_copy(z_sh.at[qidx_v], h0_v.at[pl.ds(0, QC)])
        pltpu.sync_copy(h0_v.at[pl.ds(0, QC)],
                        num_hbm.at[cid, pl.ds(row * QC, QC)])
        return 0
    lax.fori_loop(0, Q_PER_TILE // QC, qchunk, 0)


def _sc_stage(h, p, dst, nodes):
    mesh = plsc.VectorSubcoreMesh(core_axis_name="c", subcore_axis_name="s")
    f = functools.partial(
        pl.kernel, mesh=mesh,
        compiler_params=pltpu.CompilerParams(needs_layout_passes=False),
        out_type=[
            jax.ShapeDtypeStruct((NC, N_QUERY, EMBED), jnp.float32),
            jax.ShapeDtypeStruct((NC, NS, N_QUERY), jnp.float32),
        ],
        scratch_types=[
            pltpu.VMEM((CHUNK, EMBED), jnp.float32),
            pltpu.VMEM((CHUNK, EMBED), jnp.float32),
            pltpu.VMEM((CHUNK,), jnp.int32),
            pltpu.VMEM((CHUNK,), jnp.int32),
            pltpu.VMEM((CHUNK,), jnp.float32),
            pltpu.VMEM((CHUNK,), jnp.float32),
            pltpu.VMEM((N_NODES_PAD,), jnp.float32),
            pltpu.VMEM((QC,), jnp.int32),
            pltpu.VMEM((N_QUERY,), jnp.float32),
            pltpu.VMEM_SHARED((N_NODES_PAD, EMBED), jnp.float32),
            pltpu.SemaphoreType.DMA,
            pltpu.SemaphoreType.DMA,
            pltpu.SemaphoreType.DMA,
            pltpu.SemaphoreType.DMA,
            pltpu.SemaphoreType.DMA,
            pltpu.SemaphoreType.DMA,
            pltpu.SemaphoreType.DMA,
            pltpu.SemaphoreType.DMA,
        ],
    )(_sc_body)
    return f(h, p, dst, nodes)


def _combine_body(num_ref, den_ref, out_ref):
    n = num_ref[0] + num_ref[1]
    d = jnp.sum(den_ref[...], axis=(0, 1))
    out_ref[...] = n / (d[:, None] + 1e-16)


def _combine_stage(num, den):
    QB = 512
    return pl.pallas_call(
        _combine_body,
        grid=(N_QUERY // QB,),
        in_specs=[
            pl.BlockSpec((NC, QB, EMBED), lambda i: (0, i, 0)),
            pl.BlockSpec((NC, NS, QB), lambda i: (0, 0, i)),
        ],
        out_specs=pl.BlockSpec((QB, EMBED), lambda i: (i, 0)),
        out_shape=jax.ShapeDtypeStruct((N_QUERY, EMBED), jnp.float32),
    )(num, den)


def kernel(nodes, edge_index, embed_u, rep, W1, b1, W2, b2, W3, b3):
    dst = edge_index[1]
    b1r = b1.reshape(1, EMBED)
    b2r = b2.reshape(1, EMBED)
    b3r = b3.reshape(1, 1)
    h, p = _mlp_stage(embed_u, rep, W1, b1r, W2, b2r, W3, b3r)
    num, den = _sc_stage(h, p, dst.astype(jnp.int32), nodes.astype(jnp.int32))
    return _combine_stage(num, den)


# speedup vs baseline: 1.4937x; 1.1085x over previous
import functools

import jax
import jax.numpy as jnp
from jax import lax
from jax.experimental import pallas as pl
from jax.experimental.pallas import tpu as pltpu
from jax.experimental.pallas import tpu_sc as plsc

N_NODES = 10000
N_EDGES = 320000
EMBED = 128
N_QUERY = 4096

E_BLOCK = 4096

NC = 2
NS = 16
NW = NC * NS
EDGES_PER_TILE = N_EDGES // NW
CHUNK = 80
NCHUNK = EDGES_PER_TILE // CHUNK
N_NODES_PAD = 10240
ZROWS = N_NODES_PAD // NS
QC = 64
NQC = N_QUERY // QC
Q_PER_TILE = N_QUERY // NS


def _mlp_body(u_ref, rep_ref, w1_ref, b1_ref, w2_ref, b2_ref,
              w3_ref, b3_ref, ei_ref, h_ref, p_ref, dst_ref):
    u = u_ref[...]
    dst_ref[...] = ei_ref[1, :]
    x = jnp.dot(u, w1_ref[0:EMBED], preferred_element_type=jnp.float32)
    x = x + jnp.dot(rep_ref[...], w1_ref[EMBED:2 * EMBED],
                    preferred_element_type=jnp.float32)
    x = jnp.maximum(x + b1_ref[...], 0.0)
    x = jnp.maximum(jnp.dot(x, w2_ref[...], preferred_element_type=jnp.float32)
                    + b2_ref[...], 0.0)
    s = jnp.dot(x, w3_ref[...], preferred_element_type=jnp.float32) + b3_ref[...]
    p = jnp.exp(s)
    h_ref[...] = u * p
    p_ref[...] = p[:, 0]


def _mlp_stage(u, rep, w1, b1, w2, b2, w3, b3r, ei):
    grid = (N_EDGES + E_BLOCK - 1) // E_BLOCK
    return pl.pallas_call(
        _mlp_body,
        grid=(grid,),
        in_specs=[
            pl.BlockSpec((E_BLOCK, EMBED), lambda i: (i, 0)),
            pl.BlockSpec((E_BLOCK, EMBED), lambda i: (i, 0)),
            pl.BlockSpec((2 * EMBED, EMBED), lambda i: (0, 0)),
            pl.BlockSpec((1, EMBED), lambda i: (0, 0)),
            pl.BlockSpec((EMBED, EMBED), lambda i: (0, 0)),
            pl.BlockSpec((1, EMBED), lambda i: (0, 0)),
            pl.BlockSpec((EMBED, 1), lambda i: (0, 0)),
            pl.BlockSpec((1, 1), lambda i: (0, 0)),
            pl.BlockSpec((2, E_BLOCK), lambda i: (0, i)),
        ],
        out_specs=[
            pl.BlockSpec((E_BLOCK, EMBED), lambda i: (i, 0)),
            pl.BlockSpec((E_BLOCK,), lambda i: (i,)),
            pl.BlockSpec((E_BLOCK,), lambda i: (i,)),
        ],
        out_shape=[
            jax.ShapeDtypeStruct((N_EDGES, EMBED), jnp.float32),
            jax.ShapeDtypeStruct((N_EDGES,), jnp.float32),
            jax.ShapeDtypeStruct((N_EDGES,), jnp.int32),
        ],
    )(u, rep, w1, b1, w2, b2, w3, b3r, ei)


def _sc_body(h_hbm, p_hbm, dst_hbm, nodes_hbm, num_hbm, den_hbm,
             h0_v, h1_v, d0_v, d1_v, p0_v, p1_v, denom_v, qidx_v, qden_v, z_sh,
             hsem0, hsem1, dsem0, dsem1, psem0, psem1, ssem0, ssem1):
    cid = lax.axis_index("c")
    sid = lax.axis_index("s")
    wid = sid * NC + cid

    zeros16 = jnp.zeros((16,), jnp.float32)

    def zero_row(r, _):
        for k in range(EMBED // 16):
            h0_v[r, pl.ds(k * 16, 16)] = zeros16
        return 0
    lax.fori_loop(0, CHUNK, zero_row, 0)

    def zero_den(i, _):
        denom_v[pl.ds(i * 16, 16)] = zeros16
        return 0
    lax.fori_loop(0, N_NODES_PAD // 16, zero_den, 0)

    def zcopy(j, _):
        pltpu.sync_copy(h0_v, z_sh.at[pl.ds(sid * ZROWS + j * CHUNK, CHUNK)])
        return 0
    lax.fori_loop(0, ZROWS // CHUNK, zcopy, 0)

    plsc.subcore_barrier()

    def _start_load(j, hbuf, dbuf, pbuf, hsem, dsem, psem):
        base = wid * EDGES_PER_TILE + j * CHUNK
        pltpu.async_copy(h_hbm.at[pl.ds(base, CHUNK)], hbuf, hsem)
        pltpu.async_copy(dst_hbm.at[pl.ds(base, CHUNK)], dbuf, dsem)
        pltpu.async_copy(p_hbm.at[pl.ds(base, CHUNK)], pbuf, psem)

    def _wait_load(j, hbuf, dbuf, pbuf, hsem, dsem, psem):
        base = wid * EDGES_PER_TILE + j * CHUNK
        pltpu.make_async_copy(h_hbm.at[pl.ds(base, CHUNK)], hbuf, hsem).wait()
        pltpu.make_async_copy(dst_hbm.at[pl.ds(base, CHUNK)], dbuf, dsem).wait()
        pltpu.make_async_copy(p_hbm.at[pl.ds(base, CHUNK)], pbuf, psem).wait()

    def _start_scatter(hbuf, dbuf, ssem):
        pltpu.async_copy(hbuf, z_sh.at[dbuf], ssem, add=True)

    def _wait_scatter(hbuf, dbuf, ssem):
        pltpu.make_async_copy(hbuf, z_sh.at[dbuf], ssem).wait()

    def _denom(dbuf, pbuf):
        for k in range(CHUNK // 16):
            dvec = dbuf[pl.ds(k * 16, 16)]
            pvec = pbuf[pl.ds(k * 16, 16)]
            plsc.addupdate_scatter(denom_v, [dvec], pvec)

    _start_load(0, h0_v, d0_v, p0_v, hsem0, dsem0, psem0)

    def chunk_pair(j2, _):
        j0 = 2 * j2
        _wait_load(j0, h0_v, d0_v, p0_v, hsem0, dsem0, psem0)

        @pl.when(j2 > 0)
        def _():
            _wait_scatter(h1_v, d1_v, ssem1)
        _start_load(j0 + 1, h1_v, d1_v, p1_v, hsem1, dsem1, psem1)
        _start_scatter(h0_v, d0_v, ssem0)
        _denom(d0_v, p0_v)
        _wait_load(j0 + 1, h1_v, d1_v, p1_v, hsem1, dsem1, psem1)
        _wait_scatter(h0_v, d0_v, ssem0)
        _start_load(j0 + 2, h0_v, d0_v, p0_v, hsem0, dsem0, psem0)
        _start_scatter(h1_v, d1_v, ssem1)
        _denom(d1_v, p1_v)
        return 0
    lax.fori_loop(0, (NCHUNK - 1) // 2, chunk_pair, 0)
    _wait_scatter(h1_v, d1_v, ssem1)
    _wait_load(NCHUNK - 1, h0_v, d0_v, p0_v, hsem0, dsem0, psem0)
    _start_scatter(h0_v, d0_v, ssem0)
    _denom(d0_v, p0_v)
    _wait_scatter(h0_v, d0_v, ssem0)

    def qden(t, _):
        pltpu.sync_copy(nodes_hbm.at[pl.ds(t * QC, QC)], qidx_v)
        for k in range(QC // 16):
            ivec = qidx_v[pl.ds(k * 16, 16)]
            qden_v[pl.ds(t * QC + k * 16, 16)] = plsc.load_gather(
                denom_v, [ivec])
        return 0
    lax.fori_loop(0, NQC, qden, 0)
    pltpu.sync_copy(qden_v, den_hbm.at[cid, sid])

    plsc.subcore_barrier()

    def qchunk(t, _):
        row = sid * (Q_PER_TILE // QC) + t
        pltpu.sync_copy(nodes_hbm.at[pl.ds(row * QC, QC)], qidx_v)
        pltpu.sync_copy(z_sh.at[qidx_v], h0_v.at[pl.ds(0, QC)])
        pltpu.sync_copy(h0_v.at[pl.ds(0, QC)],
                        num_hbm.at[cid, pl.ds(row * QC, QC)])
        return 0
    lax.fori_loop(0, Q_PER_TILE // QC, qchunk, 0)


def _sc_stage(h, p, dst, nodes):
    mesh = plsc.VectorSubcoreMesh(core_axis_name="c", subcore_axis_name="s")
    f = functools.partial(
        pl.kernel, mesh=mesh,
        compiler_params=pltpu.CompilerParams(needs_layout_passes=False),
        out_type=[
            jax.ShapeDtypeStruct((NC, N_QUERY, EMBED), jnp.float32),
            jax.ShapeDtypeStruct((NC, NS, N_QUERY), jnp.float32),
        ],
        scratch_types=[
            pltpu.VMEM((CHUNK, EMBED), jnp.float32),
            pltpu.VMEM((CHUNK, EMBED), jnp.float32),
            pltpu.VMEM((CHUNK,), jnp.int32),
            pltpu.VMEM((CHUNK,), jnp.int32),
            pltpu.VMEM((CHUNK,), jnp.float32),
            pltpu.VMEM((CHUNK,), jnp.float32),
            pltpu.VMEM((N_NODES_PAD,), jnp.float32),
            pltpu.VMEM((QC,), jnp.int32),
            pltpu.VMEM((N_QUERY,), jnp.float32),
            pltpu.VMEM_SHARED((N_NODES_PAD, EMBED), jnp.float32),
            pltpu.SemaphoreType.DMA,
            pltpu.SemaphoreType.DMA,
            pltpu.SemaphoreType.DMA,
            pltpu.SemaphoreType.DMA,
            pltpu.SemaphoreType.DMA,
            pltpu.SemaphoreType.DMA,
            pltpu.SemaphoreType.DMA,
            pltpu.SemaphoreType.DMA,
        ],
    )(_sc_body)
    return f(h, p, dst, nodes)


def _combine_body(num_ref, den_ref, out_ref):
    n = num_ref[0] + num_ref[1]
    d = jnp.sum(den_ref[...], axis=(0, 1))
    out_ref[...] = n / (d[:, None] + 1e-16)


def _combine_stage(num, den):
    QB = 512
    return pl.pallas_call(
        _combine_body,
        grid=(N_QUERY // QB,),
        in_specs=[
            pl.BlockSpec((NC, QB, EMBED), lambda i: (0, i, 0)),
            pl.BlockSpec((NC, NS, QB), lambda i: (0, 0, i)),
        ],
        out_specs=pl.BlockSpec((QB, EMBED), lambda i: (i, 0)),
        out_shape=jax.ShapeDtypeStruct((N_QUERY, EMBED), jnp.float32),
    )(num, den)


def kernel(nodes, edge_index, embed_u, rep, W1, b1, W2, b2, W3, b3):
    b1r = b1.reshape(1, EMBED)
    b2r = b2.reshape(1, EMBED)
    b3r = b3.reshape(1, 1)
    h, p, dst = _mlp_stage(embed_u, rep, W1, b1r, W2, b2r, W3, b3r,
                           edge_index.astype(jnp.int32))
    num, den = _sc_stage(h, p, dst, nodes.astype(jnp.int32))
    return _combine_stage(num, den)


# speedup vs baseline: 1.5319x; 1.0256x over previous
import functools

import jax
import jax.numpy as jnp
from jax import lax
from jax.experimental import pallas as pl
from jax.experimental.pallas import tpu as pltpu
from jax.experimental.pallas import tpu_sc as plsc

N_NODES = 10000
N_EDGES = 320000
EMBED = 128
N_QUERY = 4096

E_BLOCK = 8192

NC = 2
NS = 16
NW = NC * NS
EDGES_PER_TILE = N_EDGES // NW
CHUNK = 80
NCHUNK = EDGES_PER_TILE // CHUNK
N_NODES_PAD = 10240
ZROWS = N_NODES_PAD // NS
QC = 64
NQC = N_QUERY // QC
Q_PER_TILE = N_QUERY // NS


def _mlp_body(u_ref, rep_ref, w1_ref, b1_ref, w2_ref, b2_ref,
              w3_ref, b3_ref, ei_ref, h_ref, p_ref, dst_ref):
    u = u_ref[...]
    dst_ref[...] = ei_ref[1, :]
    x = jnp.dot(u, w1_ref[0:EMBED], preferred_element_type=jnp.float32)
    x = x + jnp.dot(rep_ref[...], w1_ref[EMBED:2 * EMBED],
                    preferred_element_type=jnp.float32)
    x = jnp.maximum(x + b1_ref[...], 0.0)
    x = jnp.maximum(jnp.dot(x, w2_ref[...], preferred_element_type=jnp.float32)
                    + b2_ref[...], 0.0)
    s = jnp.dot(x, w3_ref[...], preferred_element_type=jnp.float32) + b3_ref[...]
    p = jnp.exp(s)
    h_ref[...] = u * p
    p_ref[...] = p[:, 0]


def _mlp_stage(u, rep, w1, b1, w2, b2, w3, b3r, ei):
    grid = (N_EDGES + E_BLOCK - 1) // E_BLOCK
    return pl.pallas_call(
        _mlp_body,
        grid=(grid,),
        in_specs=[
            pl.BlockSpec((E_BLOCK, EMBED), lambda i: (i, 0)),
            pl.BlockSpec((E_BLOCK, EMBED), lambda i: (i, 0)),
            pl.BlockSpec((2 * EMBED, EMBED), lambda i: (0, 0)),
            pl.BlockSpec((1, EMBED), lambda i: (0, 0)),
            pl.BlockSpec((EMBED, EMBED), lambda i: (0, 0)),
            pl.BlockSpec((1, EMBED), lambda i: (0, 0)),
            pl.BlockSpec((EMBED, 1), lambda i: (0, 0)),
            pl.BlockSpec((1, 1), lambda i: (0, 0)),
            pl.BlockSpec((2, E_BLOCK), lambda i: (0, i)),
        ],
        out_specs=[
            pl.BlockSpec((E_BLOCK, EMBED), lambda i: (i, 0)),
            pl.BlockSpec((E_BLOCK,), lambda i: (i,)),
            pl.BlockSpec((E_BLOCK,), lambda i: (i,)),
        ],
        out_shape=[
            jax.ShapeDtypeStruct((N_EDGES, EMBED), jnp.float32),
            jax.ShapeDtypeStruct((N_EDGES,), jnp.float32),
            jax.ShapeDtypeStruct((N_EDGES,), jnp.int32),
        ],
    )(u, rep, w1, b1, w2, b2, w3, b3r, ei)


def _sc_body(h_hbm, p_hbm, dst_hbm, nodes_hbm, num_hbm, den_hbm,
             h0_v, h1_v, d0_v, d1_v, p0_v, p1_v, denom_v, qidx_v, qden_v, z_sh,
             hsem0, hsem1, dsem0, dsem1, psem0, psem1, ssem0, ssem1):
    cid = lax.axis_index("c")
    sid = lax.axis_index("s")
    wid = sid * NC + cid

    zeros16 = jnp.zeros((16,), jnp.float32)

    def zero_row(r, _):
        for k in range(EMBED // 16):
            h0_v[r, pl.ds(k * 16, 16)] = zeros16
        return 0
    lax.fori_loop(0, CHUNK, zero_row, 0)

    def zero_den(i, _):
        denom_v[pl.ds(i * 16, 16)] = zeros16
        return 0
    lax.fori_loop(0, N_NODES_PAD // 16, zero_den, 0)

    def zcopy(j, _):
        pltpu.sync_copy(h0_v, z_sh.at[pl.ds(sid * ZROWS + j * CHUNK, CHUNK)])
        return 0
    lax.fori_loop(0, ZROWS // CHUNK, zcopy, 0)

    plsc.subcore_barrier()

    def _start_load(j, hbuf, dbuf, pbuf, hsem, dsem, psem):
        base = wid * EDGES_PER_TILE + j * CHUNK
        pltpu.async_copy(h_hbm.at[pl.ds(base, CHUNK)], hbuf, hsem)
        pltpu.async_copy(dst_hbm.at[pl.ds(base, CHUNK)], dbuf, dsem)
        pltpu.async_copy(p_hbm.at[pl.ds(base, CHUNK)], pbuf, psem)

    def _wait_load(j, hbuf, dbuf, pbuf, hsem, dsem, psem):
        base = wid * EDGES_PER_TILE + j * CHUNK
        pltpu.make_async_copy(h_hbm.at[pl.ds(base, CHUNK)], hbuf, hsem).wait()
        pltpu.make_async_copy(dst_hbm.at[pl.ds(base, CHUNK)], dbuf, dsem).wait()
        pltpu.make_async_copy(p_hbm.at[pl.ds(base, CHUNK)], pbuf, psem).wait()

    def _start_scatter(hbuf, dbuf, ssem):
        pltpu.async_copy(hbuf, z_sh.at[dbuf], ssem, add=True)

    def _wait_scatter(hbuf, dbuf, ssem):
        pltpu.make_async_copy(hbuf, z_sh.at[dbuf], ssem).wait()

    def _denom(dbuf, pbuf):
        for k in range(CHUNK // 16):
            dvec = dbuf[pl.ds(k * 16, 16)]
            pvec = pbuf[pl.ds(k * 16, 16)]
            plsc.addupdate_scatter(denom_v, [dvec], pvec)

    _start_load(0, h0_v, d0_v, p0_v, hsem0, dsem0, psem0)

    def chunk_pair(j2, _):
        j0 = 2 * j2
        _wait_load(j0, h0_v, d0_v, p0_v, hsem0, dsem0, psem0)

        @pl.when(j2 > 0)
        def _():
            _wait_scatter(h1_v, d1_v, ssem1)
        _start_load(j0 + 1, h1_v, d1_v, p1_v, hsem1, dsem1, psem1)
        _start_scatter(h0_v, d0_v, ssem0)
        _denom(d0_v, p0_v)
        _wait_load(j0 + 1, h1_v, d1_v, p1_v, hsem1, dsem1, psem1)
        _wait_scatter(h0_v, d0_v, ssem0)
        _start_load(j0 + 2, h0_v, d0_v, p0_v, hsem0, dsem0, psem0)
        _start_scatter(h1_v, d1_v, ssem1)
        _denom(d1_v, p1_v)
        return 0
    lax.fori_loop(0, (NCHUNK - 1) // 2, chunk_pair, 0)
    _wait_scatter(h1_v, d1_v, ssem1)
    _wait_load(NCHUNK - 1, h0_v, d0_v, p0_v, hsem0, dsem0, psem0)
    _start_scatter(h0_v, d0_v, ssem0)
    _denom(d0_v, p0_v)
    _wait_scatter(h0_v, d0_v, ssem0)

    def qden(t, _):
        pltpu.sync_copy(nodes_hbm.at[pl.ds(t * QC, QC)], qidx_v)
        for k in range(QC // 16):
            ivec = qidx_v[pl.ds(k * 16, 16)]
            qden_v[pl.ds(t * QC + k * 16, 16)] = plsc.load_gather(
                denom_v, [ivec])
        return 0
    lax.fori_loop(0, NQC, qden, 0)
    pltpu.sync_copy(qden_v, den_hbm.at[cid, sid])

    plsc.subcore_barrier()

    def qchunk(t, _):
        row = sid * (Q_PER_TILE // QC) + t
        pltpu.sync_copy(nodes_hbm.at[pl.ds(row * QC, QC)], qidx_v)
        pltpu.sync_copy(z_sh.at[qidx_v], h0_v.at[pl.ds(0, QC)])
        pltpu.sync_copy(h0_v.at[pl.ds(0, QC)],
                        num_hbm.at[cid, pl.ds(row * QC, QC)])
        return 0
    lax.fori_loop(0, Q_PER_TILE // QC, qchunk, 0)


def _sc_stage(h, p, dst, nodes):
    mesh = plsc.VectorSubcoreMesh(core_axis_name="c", subcore_axis_name="s")
    f = functools.partial(
        pl.kernel, mesh=mesh,
        compiler_params=pltpu.CompilerParams(needs_layout_passes=False),
        out_type=[
            jax.ShapeDtypeStruct((NC, N_QUERY, EMBED), jnp.float32),
            jax.ShapeDtypeStruct((NC, NS, N_QUERY), jnp.float32),
        ],
        scratch_types=[
            pltpu.VMEM((CHUNK, EMBED), jnp.float32),
            pltpu.VMEM((CHUNK, EMBED), jnp.float32),
            pltpu.VMEM((CHUNK,), jnp.int32),
            pltpu.VMEM((CHUNK,), jnp.int32),
            pltpu.VMEM((CHUNK,), jnp.float32),
            pltpu.VMEM((CHUNK,), jnp.float32),
            pltpu.VMEM((N_NODES_PAD,), jnp.float32),
            pltpu.VMEM((QC,), jnp.int32),
            pltpu.VMEM((N_QUERY,), jnp.float32),
            pltpu.VMEM_SHARED((N_NODES_PAD, EMBED), jnp.float32),
            pltpu.SemaphoreType.DMA,
            pltpu.SemaphoreType.DMA,
            pltpu.SemaphoreType.DMA,
            pltpu.SemaphoreType.DMA,
            pltpu.SemaphoreType.DMA,
            pltpu.SemaphoreType.DMA,
            pltpu.SemaphoreType.DMA,
            pltpu.SemaphoreType.DMA,
        ],
    )(_sc_body)
    return f(h, p, dst, nodes)


def _combine_body(num_ref, den_ref, out_ref):
    n = num_ref[0] + num_ref[1]
    d = jnp.sum(den_ref[...], axis=(0, 1))
    out_ref[...] = n / (d[:, None] + 1e-16)


def _combine_stage(num, den):
    QB = 512
    return pl.pallas_call(
        _combine_body,
        grid=(N_QUERY // QB,),
        in_specs=[
            pl.BlockSpec((NC, QB, EMBED), lambda i: (0, i, 0)),
            pl.BlockSpec((NC, NS, QB), lambda i: (0, 0, i)),
        ],
        out_specs=pl.BlockSpec((QB, EMBED), lambda i: (i, 0)),
        out_shape=jax.ShapeDtypeStruct((N_QUERY, EMBED), jnp.float32),
    )(num, den)


def kernel(nodes, edge_index, embed_u, rep, W1, b1, W2, b2, W3, b3):
    b1r = b1.reshape(1, EMBED)
    b2r = b2.reshape(1, EMBED)
    b3r = b3.reshape(1, 1)
    h, p, dst = _mlp_stage(embed_u, rep, W1, b1r, W2, b2r, W3, b3r,
                           edge_index.astype(jnp.int32))
    num, den = _sc_stage(h, p, dst, nodes.astype(jnp.int32))
    return _combine_stage(num, den)
